# Initial kernel scaffold; baseline (speedup 1.0000x reference)
import functools

import jax
import jax.numpy as jnp
from jax import lax
from jax.experimental import pallas as pl
from jax.experimental.pallas import tpu as pltpu
from jax.experimental.pallas import tpu_sc as plsc

N = 10000
H = 256
HH = 128
E = 160000
L = 160000

NC = 2
NS = 16
NW = NC * NS

NP = 10240
RPT = NP // NS

EPT = E // NS
ECH = 80
ENIT = EPT // ECH

PPW = L // NW
PCH = 40
PNIT = PPW // PCH

BN = 1024

_MESH = plsc.VectorSubcoreMesh(
    core_axis_name="c", subcore_axis_name="s", num_cores=NC, num_subcores=NS)


def _dotT(x, w):
    return lax.dot_general(x, w, (((1,), (1,)), ((), ())),
                           preferred_element_type=jnp.float32)


def _tc_a(emb_p, W1l, W1r, b1l):
    def body(x_ref, wl_ref, wr_ref, b_ref, yL_ref, yR_ref, h_ref):
        x = x_ref[...]
        y = _dotT(x, wl_ref[...])
        yL_ref[...] = y[:, :HH]
        yR_ref[...] = y[:, HH:]
        h_ref[...] = _dotT(x, wr_ref[...]) + b_ref[...]

    return pl.pallas_call(
        body,
        grid=(NP // BN,),
        in_specs=[
            pl.BlockSpec((BN, H), lambda i: (i, 0)),
            pl.BlockSpec((H, H), lambda i: (0, 0)),
            pl.BlockSpec((H, H), lambda i: (0, 0)),
            pl.BlockSpec((1, H), lambda i: (0, 0)),
        ],
        out_specs=[
            pl.BlockSpec((BN, HH), lambda i: (i, 0)),
            pl.BlockSpec((BN, HH), lambda i: (i, 0)),
            pl.BlockSpec((BN, H), lambda i: (i, 0)),
        ],
        out_shape=[
            jax.ShapeDtypeStruct((NP, HH), jnp.float32),
            jax.ShapeDtypeStruct((NP, HH), jnp.float32),
            jax.ShapeDtypeStruct((NP, H), jnp.float32),
        ],
    )(emb_p, W1l, W1r, b1l)


def _tc_b(s1L, s1R, cnt, h1, W2l, W2r, b2l):
    def body(sL_ref, sR_ref, cnt_ref, h_ref, wl_ref, wr_ref, b_ref,
             yL_ref, yR_ref, h2_ref):
        inv = 1.0 / jnp.maximum(cnt_ref[...], 1.0)
        s = jnp.concatenate([sL_ref[...], sR_ref[...]], axis=1)
        x = jnp.maximum(s * inv + h_ref[...], 0.0)
        y = _dotT(x, wl_ref[...])
        yL_ref[...] = y[:, :HH]
        yR_ref[...] = y[:, HH:]
        h2_ref[...] = _dotT(x, wr_ref[...]) + b_ref[...]

    return pl.pallas_call(
        body,
        grid=(NP // BN,),
        in_specs=[
            pl.BlockSpec((BN, HH), lambda i: (i, 0)),
            pl.BlockSpec((BN, HH), lambda i: (i, 0)),
            pl.BlockSpec((BN, 1), lambda i: (i, 0)),
            pl.BlockSpec((BN, H), lambda i: (i, 0)),
            pl.BlockSpec((H, H), lambda i: (0, 0)),
            pl.BlockSpec((H, H), lambda i: (0, 0)),
            pl.BlockSpec((1, H), lambda i: (0, 0)),
        ],
        out_specs=[
            pl.BlockSpec((BN, HH), lambda i: (i, 0)),
            pl.BlockSpec((BN, HH), lambda i: (i, 0)),
            pl.BlockSpec((BN, H), lambda i: (i, 0)),
        ],
        out_shape=[
            jax.ShapeDtypeStruct((NP, HH), jnp.float32),
            jax.ShapeDtypeStruct((NP, HH), jnp.float32),
            jax.ShapeDtypeStruct((NP, H), jnp.float32),
        ],
    )(s1L, s1R, cnt, h1, W2l, W2r, b2l)


def _tc_c(s2L, s2R, cnt, h2):
    def body(sL_ref, sR_ref, cnt_ref, h_ref, zL_ref, zR_ref):
        inv = 1.0 / jnp.maximum(cnt_ref[...], 1.0)
        h = h_ref[...]
        zL_ref[...] = sL_ref[...] * inv + h[:, :HH]
        zR_ref[...] = sR_ref[...] * inv + h[:, HH:]

    return pl.pallas_call(
        body,
        grid=(NP // BN,),
        in_specs=[
            pl.BlockSpec((BN, HH), lambda i: (i, 0)),
            pl.BlockSpec((BN, HH), lambda i: (i, 0)),
            pl.BlockSpec((BN, 1), lambda i: (i, 0)),
            pl.BlockSpec((BN, H), lambda i: (i, 0)),
        ],
        out_specs=[
            pl.BlockSpec((BN, HH), lambda i: (i, 0)),
            pl.BlockSpec((BN, HH), lambda i: (i, 0)),
        ],
        out_shape=[
            jax.ShapeDtypeStruct((NP, HH), jnp.float32),
            jax.ShapeDtypeStruct((NP, HH), jnp.float32),
        ],
    )(s2L, s2R, cnt, h2)


def _make_seg(with_cnt):
    out_type = [
        jax.ShapeDtypeStruct((NP, HH), jnp.float32),
        jax.ShapeDtypeStruct((NP, HH), jnp.float32),
    ]
    if with_cnt:
        out_type.append(jax.ShapeDtypeStruct((NP,), jnp.float32))

    scratch = [
        pltpu.VMEM((ECH,), jnp.int32),
        pltpu.VMEM((ECH,), jnp.int32),
        pltpu.VMEM((ECH, HH), jnp.float32),
        pltpu.VMEM((16, HH), jnp.float32),
        pltpu.VMEM((NP,), jnp.float32),
        pltpu.VMEM((RPT,), jnp.float32),
        pltpu.VMEM((RPT,), jnp.float32),
        pltpu.VMEM_SHARED((NP, HH), jnp.float32),
        pltpu.VMEM_SHARED((NS, NP), jnp.float32),
    ]

    def body(src_hbm, dst_hbm, yl_hbm, yr_hbm, *rest):
        if with_cnt:
            outL, outR, cnt_hbm = rest[:3]
            scr = rest[3:]
        else:
            outL, outR = rest[:2]
            scr = rest[2:]
        (sidx_v, didx_v, rows_v, zbuf, cnt_local, red_a, red_b,
         acc, cnt_stage) = scr

        c = lax.axis_index("c")
        s = lax.axis_index("s")
        row0 = s * RPT

        zeros16 = jnp.zeros((16,), jnp.float32)
        ones16 = jnp.full((16,), 1.0, jnp.float32)

        for r in range(16):
            for j in range(HH // 16):
                zbuf[r, pl.ds(j * 16, 16)] = zeros16

        def zacc(i, carry):
            pltpu.sync_copy(zbuf, acc.at[pl.ds(row0 + i * 16, 16)])
            return carry
        lax.fori_loop(0, RPT // 16, zacc, 0)

        if with_cnt:
            def zcnt(i, carry):
                cnt_local[pl.ds(i * 16, 16)] = zeros16
                return carry
            lax.fori_loop(0, NP // 16, zcnt, 0)

        plsc.subcore_barrier()

        ebase = s * EPT

        def eloop(i, carry):
            base = ebase + i * ECH
            pltpu.sync_copy(src_hbm.at[pl.ds(base, ECH)], sidx_v)
            pltpu.sync_copy(dst_hbm.at[pl.ds(base, ECH)], didx_v)

            @pl.when(c == 0)
            def _():
                pltpu.sync_copy(yl_hbm.at[sidx_v], rows_v)

            @pl.when(c == 1)
            def _():
                pltpu.sync_copy(yr_hbm.at[sidx_v], rows_v)

            if with_cnt:
                @pl.when(c == 0)
                def _():
                    for j in range(ECH // 16):
                        idx16 = didx_v[pl.ds(j * 16, 16)]
                        plsc.addupdate_scatter(cnt_local, [idx16], ones16)

            pltpu.sync_copy(rows_v, acc.at[didx_v], add=True)
            return carry
        lax.fori_loop(0, ENIT, eloop, 0)

        plsc.subcore_barrier()

        @pl.when(c == 0)
        def _():
            pltpu.sync_copy(acc.at[pl.ds(row0, RPT)], outL.at[pl.ds(row0, RPT)])

        @pl.when(c == 1)
        def _():
            pltpu.sync_copy(acc.at[pl.ds(row0, RPT)], outR.at[pl.ds(row0, RPT)])

        if with_cnt:
            @pl.when(c == 0)
            def _():
                pltpu.sync_copy(cnt_local, cnt_stage.at[s])

            plsc.subcore_barrier()

            @pl.when(c == 0)
            def _():
                pltpu.sync_copy(cnt_stage.at[0, pl.ds(row0, RPT)], red_a)
                for t in range(1, NS):
                    pltpu.sync_copy(cnt_stage.at[t, pl.ds(row0, RPT)], red_b)
                    for j in range(RPT // 16):
                        red_a[pl.ds(j * 16, 16)] = (
                            red_a[pl.ds(j * 16, 16)] + red_b[pl.ds(j * 16, 16)])
                pltpu.sync_copy(red_a, cnt_hbm.at[pl.ds(row0, RPT)])

    return pl.kernel(body, out_type=out_type, mesh=_MESH,
                     scratch_types=scratch)


_seg_with_cnt = _make_seg(True)
_seg_no_cnt = _make_seg(False)


def _sc_decode(lsrc, ldst, zL, zR):
    out_type = jax.ShapeDtypeStruct((L,), jnp.float32)
    scratch = [
        pltpu.VMEM((PCH,), jnp.int32),
        pltpu.VMEM((PCH,), jnp.int32),
        pltpu.VMEM((PCH, HH), jnp.float32),
        pltpu.VMEM((PCH, HH), jnp.float32),
        pltpu.VMEM((PCH, HH), jnp.float32),
        pltpu.VMEM((PCH, HH), jnp.float32),
        pltpu.VMEM((PCH,), jnp.float32),
    ]

    def body(ls_hbm, ld_hbm, zl_hbm, zr_hbm, out_hbm,
             sidx_v, didx_v, aL, aR, bL, bR, out_v):
        c = lax.axis_index("c")
        s = lax.axis_index("s")
        wid = s * NC + c
        wbase = wid * PPW

        def ploop(i, carry):
            base = wbase + i * PCH
            pltpu.sync_copy(ls_hbm.at[pl.ds(base, PCH)], sidx_v)
            pltpu.sync_copy(ld_hbm.at[pl.ds(base, PCH)], didx_v)
            pltpu.sync_copy(zl_hbm.at[sidx_v], aL)
            pltpu.sync_copy(zr_hbm.at[sidx_v], aR)
            pltpu.sync_copy(zl_hbm.at[didx_v], bL)
            pltpu.sync_copy(zr_hbm.at[didx_v], bR)

            def pair(p, carry2):
                dot = aL[p, pl.ds(0, 16)] * bL[p, pl.ds(0, 16)]
                for j in range(1, HH // 16):
                    dot = dot + aL[p, pl.ds(16 * j, 16)] * bL[p, pl.ds(16 * j, 16)]
                for j in range(HH // 16):
                    dot = dot + aR[p, pl.ds(16 * j, 16)] * bR[p, pl.ds(16 * j, 16)]
                out_v[p] = jnp.sum(dot)
                return carry2
            lax.fori_loop(0, PCH, pair, 0)

            pltpu.sync_copy(out_v, out_hbm.at[pl.ds(base, PCH)])
            return carry
        lax.fori_loop(0, PNIT, ploop, 0)

    return pl.kernel(body, out_type=out_type, mesh=_MESH,
                     scratch_types=scratch)(lsrc, ldst, zL, zR)


def kernel(edge_index, edge_label_index, emb, W1l, b1l, W1r, W2l, b2l, W2r):
    src = edge_index[0]
    dst = edge_index[1]
    lsrc = edge_label_index[0]
    ldst = edge_label_index[1]

    emb_p = jnp.concatenate(
        [emb, jnp.zeros((NP - N, H), emb.dtype)], axis=0)
    b1 = b1l.reshape(1, H)
    b2 = b2l.reshape(1, H)

    y1L, y1R, h1 = _tc_a(emb_p, W1l, W1r, b1)
    s1L, s1R, cnt = _seg_with_cnt(src, dst, y1L, y1R)
    cnt2 = cnt.reshape(NP, 1)
    y2L, y2R, h2 = _tc_b(s1L, s1R, cnt2, h1, W2l, W2r, b2)
    s2L, s2R = _seg_no_cnt(src, dst, y2L, y2R)
    zL, zR = _tc_c(s2L, s2R, cnt2, h2)
    return _sc_decode(lsrc, ldst, zL, zR)


# speedup vs baseline: 2.1256x; 2.1256x over previous
import functools

import jax
import jax.numpy as jnp
from jax import lax
from jax.experimental import pallas as pl
from jax.experimental.pallas import tpu as pltpu
from jax.experimental.pallas import tpu_sc as plsc

N = 10000
H = 256
HH = 128
E = 160000
L = 160000

NC = 2
NS = 16
NW = NC * NS

NP = 10240
RPT = NP // NS

EPT = E // NS
ECH = 80
ENIT = EPT // ECH

PPW = L // NW
PCH = 40
PNIT = PPW // PCH

BN = 1024

_MESH = plsc.VectorSubcoreMesh(
    core_axis_name="c", subcore_axis_name="s", num_cores=NC, num_subcores=NS)


def _dotT(x, w):
    return lax.dot_general(x, w, (((1,), (1,)), ((), ())),
                           preferred_element_type=jnp.float32)


def _tc_a(emb_p, W1l, W1r, b1l):
    def body(x_ref, wl_ref, wr_ref, b_ref, yL_ref, yR_ref, h_ref):
        x = x_ref[...]
        y = _dotT(x, wl_ref[...])
        yL_ref[...] = y[:, :HH]
        yR_ref[...] = y[:, HH:]
        h_ref[...] = _dotT(x, wr_ref[...]) + b_ref[...]

    return pl.pallas_call(
        body,
        grid=(NP // BN,),
        in_specs=[
            pl.BlockSpec((BN, H), lambda i: (i, 0)),
            pl.BlockSpec((H, H), lambda i: (0, 0)),
            pl.BlockSpec((H, H), lambda i: (0, 0)),
            pl.BlockSpec((1, H), lambda i: (0, 0)),
        ],
        out_specs=[
            pl.BlockSpec((BN, HH), lambda i: (i, 0)),
            pl.BlockSpec((BN, HH), lambda i: (i, 0)),
            pl.BlockSpec((BN, H), lambda i: (i, 0)),
        ],
        out_shape=[
            jax.ShapeDtypeStruct((NP, HH), jnp.float32),
            jax.ShapeDtypeStruct((NP, HH), jnp.float32),
            jax.ShapeDtypeStruct((NP, H), jnp.float32),
        ],
    )(emb_p, W1l, W1r, b1l)


def _tc_b(s1L, s1R, cnt, h1, W2l, W2r, b2l):
    def body(sL_ref, sR_ref, cnt_ref, h_ref, wl_ref, wr_ref, b_ref,
             yL_ref, yR_ref, h2_ref):
        inv = 1.0 / jnp.maximum(cnt_ref[...], 1.0)
        s = jnp.concatenate([sL_ref[...], sR_ref[...]], axis=1)
        x = jnp.maximum(s * inv + h_ref[...], 0.0)
        y = _dotT(x, wl_ref[...])
        yL_ref[...] = y[:, :HH]
        yR_ref[...] = y[:, HH:]
        h2_ref[...] = _dotT(x, wr_ref[...]) + b_ref[...]

    return pl.pallas_call(
        body,
        grid=(NP // BN,),
        in_specs=[
            pl.BlockSpec((BN, HH), lambda i: (i, 0)),
            pl.BlockSpec((BN, HH), lambda i: (i, 0)),
            pl.BlockSpec((BN, 1), lambda i: (i, 0)),
            pl.BlockSpec((BN, H), lambda i: (i, 0)),
            pl.BlockSpec((H, H), lambda i: (0, 0)),
            pl.BlockSpec((H, H), lambda i: (0, 0)),
            pl.BlockSpec((1, H), lambda i: (0, 0)),
        ],
        out_specs=[
            pl.BlockSpec((BN, HH), lambda i: (i, 0)),
            pl.BlockSpec((BN, HH), lambda i: (i, 0)),
            pl.BlockSpec((BN, H), lambda i: (i, 0)),
        ],
        out_shape=[
            jax.ShapeDtypeStruct((NP, HH), jnp.float32),
            jax.ShapeDtypeStruct((NP, HH), jnp.float32),
            jax.ShapeDtypeStruct((NP, H), jnp.float32),
        ],
    )(s1L, s1R, cnt, h1, W2l, W2r, b2l)


def _tc_c(s2L, s2R, cnt, h2):
    def body(sL_ref, sR_ref, cnt_ref, h_ref, zL_ref, zR_ref):
        inv = 1.0 / jnp.maximum(cnt_ref[...], 1.0)
        h = h_ref[...]
        zL_ref[...] = sL_ref[...] * inv + h[:, :HH]
        zR_ref[...] = sR_ref[...] * inv + h[:, HH:]

    return pl.pallas_call(
        body,
        grid=(NP // BN,),
        in_specs=[
            pl.BlockSpec((BN, HH), lambda i: (i, 0)),
            pl.BlockSpec((BN, HH), lambda i: (i, 0)),
            pl.BlockSpec((BN, 1), lambda i: (i, 0)),
            pl.BlockSpec((BN, H), lambda i: (i, 0)),
        ],
        out_specs=[
            pl.BlockSpec((BN, HH), lambda i: (i, 0)),
            pl.BlockSpec((BN, HH), lambda i: (i, 0)),
        ],
        out_shape=[
            jax.ShapeDtypeStruct((NP, HH), jnp.float32),
            jax.ShapeDtypeStruct((NP, HH), jnp.float32),
        ],
    )(s2L, s2R, cnt, h2)


def _make_seg(with_cnt):
    out_type = [
        jax.ShapeDtypeStruct((NP, HH), jnp.float32),
        jax.ShapeDtypeStruct((NP, HH), jnp.float32),
    ]
    if with_cnt:
        out_type.append(jax.ShapeDtypeStruct((NP,), jnp.float32))

    scratch = [
        pltpu.VMEM((ECH,), jnp.int32),
        pltpu.VMEM((ECH,), jnp.int32),
        pltpu.VMEM((ECH, HH), jnp.float32),
        pltpu.VMEM((16, HH), jnp.float32),
        pltpu.VMEM((ECH,), jnp.float32),
        pltpu.VMEM_SHARED((NP, HH), jnp.float32),
        pltpu.VMEM_SHARED((NP,), jnp.float32),
    ]

    def body(src_hbm, dst_hbm, yl_hbm, yr_hbm, *rest):
        if with_cnt:
            outL, outR, cnt_hbm = rest[:3]
            scr = rest[3:]
        else:
            outL, outR = rest[:2]
            scr = rest[2:]
        (sidx_v, didx_v, rows_v, zbuf, ones_v, acc, cnt_sh) = scr

        c = lax.axis_index("c")
        s = lax.axis_index("s")
        row0 = s * RPT

        zeros16 = jnp.zeros((16,), jnp.float32)
        ones16 = jnp.full((16,), 1.0, jnp.float32)

        for r in range(16):
            for j in range(HH // 16):
                zbuf[r, pl.ds(j * 16, 16)] = zeros16
        for j in range(ECH // 16):
            ones_v[pl.ds(j * 16, 16)] = ones16

        def zacc(i, carry):
            pltpu.sync_copy(zbuf, acc.at[pl.ds(row0 + i * 16, 16)])
            return carry
        lax.fori_loop(0, RPT // 16, zacc, 0)

        if with_cnt:
            @pl.when(c == 0)
            def _():
                def zcnt(i, carry):
                    pltpu.sync_copy(zbuf.at[0], cnt_sh.at[pl.ds(row0 + i * HH, HH)])
                    return carry
                lax.fori_loop(0, RPT // HH, zcnt, 0)

        plsc.subcore_barrier()

        ebase = s * EPT

        def eloop(i, carry):
            base = ebase + i * ECH
            pltpu.sync_copy(src_hbm.at[pl.ds(base, ECH)], sidx_v)
            pltpu.sync_copy(dst_hbm.at[pl.ds(base, ECH)], didx_v)

            @pl.when(c == 0)
            def _():
                pltpu.sync_copy(yl_hbm.at[sidx_v], rows_v)

            @pl.when(c == 1)
            def _():
                pltpu.sync_copy(yr_hbm.at[sidx_v], rows_v)

            if with_cnt:
                @pl.when(c == 0)
                def _():
                    pltpu.sync_copy(ones_v, cnt_sh.at[didx_v], add=True)

            pltpu.sync_copy(rows_v, acc.at[didx_v], add=True)
            return carry
        lax.fori_loop(0, ENIT, eloop, 0)

        plsc.subcore_barrier()

        @pl.when(c == 0)
        def _():
            pltpu.sync_copy(acc.at[pl.ds(row0, RPT)], outL.at[pl.ds(row0, RPT)])

        @pl.when(c == 1)
        def _():
            pltpu.sync_copy(acc.at[pl.ds(row0, RPT)], outR.at[pl.ds(row0, RPT)])

        if with_cnt:
            @pl.when(c == 0)
            def _():
                pltpu.sync_copy(cnt_sh.at[pl.ds(row0, RPT)],
                                cnt_hbm.at[pl.ds(row0, RPT)])

    return pl.kernel(body, out_type=out_type, mesh=_MESH,
                     scratch_types=scratch)


_seg_with_cnt = _make_seg(True)
_seg_no_cnt = _make_seg(False)


def _sc_decode(lsrc, ldst, zL, zR):
    out_type = jax.ShapeDtypeStruct((L,), jnp.float32)
    scratch = [
        pltpu.VMEM((PCH,), jnp.int32),
        pltpu.VMEM((PCH,), jnp.int32),
        pltpu.VMEM((PCH, HH), jnp.float32),
        pltpu.VMEM((PCH, HH), jnp.float32),
        pltpu.VMEM((PCH, HH), jnp.float32),
        pltpu.VMEM((PCH, HH), jnp.float32),
        pltpu.VMEM((16,), jnp.float32),
        pltpu.VMEM((PCH,), jnp.float32),
    ]

    def body(ls_hbm, ld_hbm, zl_hbm, zr_hbm, out_hbm,
             sidx_v, didx_v, aL, aR, bL, bR, dot_buf, out_v):
        c = lax.axis_index("c")
        s = lax.axis_index("s")
        wid = s * NC + c
        wbase = wid * PPW
        iota16 = lax.iota(jnp.int32, 16)
        zeros16 = jnp.zeros((16,), jnp.float32)

        def ploop(i, carry):
            base = wbase + i * PCH
            pltpu.sync_copy(ls_hbm.at[pl.ds(base, PCH)], sidx_v)
            pltpu.sync_copy(ld_hbm.at[pl.ds(base, PCH)], didx_v)
            pltpu.sync_copy(zl_hbm.at[sidx_v], aL)
            pltpu.sync_copy(zr_hbm.at[sidx_v], aR)
            pltpu.sync_copy(zl_hbm.at[didx_v], bL)
            pltpu.sync_copy(zr_hbm.at[didx_v], bR)

            def grp(g0):
                res = zeros16
                for k in range(16):
                    p = g0 + k
                    dot = aL[p, pl.ds(0, 16)] * bL[p, pl.ds(0, 16)]
                    for j in range(1, HH // 16):
                        dot = dot + (aL[p, pl.ds(16 * j, 16)]
                                     * bL[p, pl.ds(16 * j, 16)])
                    for j in range(HH // 16):
                        dot = dot + (aR[p, pl.ds(16 * j, 16)]
                                     * bR[p, pl.ds(16 * j, 16)])
                    ssum = dot[0]
                    for q in range(1, 16):
                        ssum = ssum + dot[q]
                    res = jnp.where(iota16 == k, ssum, res)
                out_v[pl.ds(g0, 16)] = res
            for g0 in (0, 16, 24):
                grp(g0)

            pltpu.sync_copy(out_v, out_hbm.at[pl.ds(base, PCH)])
            return carry
        lax.fori_loop(0, PNIT, ploop, 0)

    return pl.kernel(body, out_type=out_type, mesh=_MESH,
                     scratch_types=scratch)(lsrc, ldst, zL, zR)


def kernel(edge_index, edge_label_index, emb, W1l, b1l, W1r, W2l, b2l, W2r):
    src = edge_index[0]
    dst = edge_index[1]
    lsrc = edge_label_index[0]
    ldst = edge_label_index[1]

    emb_p = jnp.concatenate(
        [emb, jnp.zeros((NP - N, H), emb.dtype)], axis=0)
    b1 = b1l.reshape(1, H)
    b2 = b2l.reshape(1, H)

    y1L, y1R, h1 = _tc_a(emb_p, W1l, W1r, b1)
    s1L, s1R, cnt = _seg_with_cnt(src, dst, y1L, y1R)
    cnt2 = cnt.reshape(NP, 1)
    y2L, y2R, h2 = _tc_b(s1L, s1R, cnt2, h1, W2l, W2r, b2)
    s2L, s2R = _seg_no_cnt(src, dst, y2L, y2R)
    zL, zR = _tc_c(s2L, s2R, cnt2, h2)
    return _sc_decode(lsrc, ldst, zL, zR)


# speedup vs baseline: 4.9787x; 2.3423x over previous
import functools

import jax
import jax.numpy as jnp
from jax import lax
from jax.experimental import pallas as pl
from jax.experimental.pallas import tpu as pltpu
from jax.experimental.pallas import tpu_sc as plsc

N = 10000
H = 256
HH = 128
E = 160000
L = 160000

NC = 2
NS = 16
NW = NC * NS

NP = 10240
RPT = NP // NS

EPT = E // NS
ECH = 40
EBLK = 5
EBNIT = 50
ERING = 5
CW = 128

LP = 163840
PPW = L // NS
PCH = 40
PNIT = PPW // PCH
PRING = 5

BN = 1024

_MESH = plsc.VectorSubcoreMesh(
    core_axis_name="c", subcore_axis_name="s", num_cores=NC, num_subcores=NS)


def _dotT(x, w):
    return lax.dot_general(x, w, (((1,), (1,)), ((), ())),
                           preferred_element_type=jnp.float32)


def _tc_a(emb_p, W1l, W1r, b1l):
    def body(x_ref, wl_ref, wr_ref, b_ref, yL_ref, yR_ref, h_ref):
        x = x_ref[...]
        y = _dotT(x, wl_ref[...])
        yL_ref[...] = y[:, :HH]
        yR_ref[...] = y[:, HH:]
        h_ref[...] = _dotT(x, wr_ref[...]) + b_ref[...]

    return pl.pallas_call(
        body,
        grid=(NP // BN,),
        in_specs=[
            pl.BlockSpec((BN, H), lambda i: (i, 0)),
            pl.BlockSpec((H, H), lambda i: (0, 0)),
            pl.BlockSpec((H, H), lambda i: (0, 0)),
            pl.BlockSpec((1, H), lambda i: (0, 0)),
        ],
        out_specs=[
            pl.BlockSpec((BN, HH), lambda i: (i, 0)),
            pl.BlockSpec((BN, HH), lambda i: (i, 0)),
            pl.BlockSpec((BN, H), lambda i: (i, 0)),
        ],
        out_shape=[
            jax.ShapeDtypeStruct((NP, HH), jnp.float32),
            jax.ShapeDtypeStruct((NP, HH), jnp.float32),
            jax.ShapeDtypeStruct((NP, H), jnp.float32),
        ],
    )(emb_p, W1l, W1r, b1l)


def _tc_b(s1L, s1R, cnt, h1, W2l, W2r, b2l):
    def body(sL_ref, sR_ref, cnt_ref, h_ref, wl_ref, wr_ref, b_ref,
             yL_ref, yR_ref, h2_ref):
        inv = 1.0 / jnp.maximum(cnt_ref[...][:, :1], 1.0)
        s = jnp.concatenate([sL_ref[...], sR_ref[...]], axis=1)
        x = jnp.maximum(s * inv + h_ref[...], 0.0)
        y = _dotT(x, wl_ref[...])
        yL_ref[...] = y[:, :HH]
        yR_ref[...] = y[:, HH:]
        h2_ref[...] = _dotT(x, wr_ref[...]) + b_ref[...]

    return pl.pallas_call(
        body,
        grid=(NP // BN,),
        in_specs=[
            pl.BlockSpec((BN, HH), lambda i: (i, 0)),
            pl.BlockSpec((BN, HH), lambda i: (i, 0)),
            pl.BlockSpec((BN, CW), lambda i: (i, 0)),
            pl.BlockSpec((BN, H), lambda i: (i, 0)),
            pl.BlockSpec((H, H), lambda i: (0, 0)),
            pl.BlockSpec((H, H), lambda i: (0, 0)),
            pl.BlockSpec((1, H), lambda i: (0, 0)),
        ],
        out_specs=[
            pl.BlockSpec((BN, HH), lambda i: (i, 0)),
            pl.BlockSpec((BN, HH), lambda i: (i, 0)),
            pl.BlockSpec((BN, H), lambda i: (i, 0)),
        ],
        out_shape=[
            jax.ShapeDtypeStruct((NP, HH), jnp.float32),
            jax.ShapeDtypeStruct((NP, HH), jnp.float32),
            jax.ShapeDtypeStruct((NP, H), jnp.float32),
        ],
    )(s1L, s1R, cnt, h1, W2l, W2r, b2l)


def _tc_c(s2L, s2R, cnt, h2):
    def body(sL_ref, sR_ref, cnt_ref, h_ref, zL_ref, zR_ref):
        inv = 1.0 / jnp.maximum(cnt_ref[...][:, :1], 1.0)
        h = h_ref[...]
        zL_ref[...] = sL_ref[...] * inv + h[:, :HH]
        zR_ref[...] = sR_ref[...] * inv + h[:, HH:]

    return pl.pallas_call(
        body,
        grid=(NP // BN,),
        in_specs=[
            pl.BlockSpec((BN, HH), lambda i: (i, 0)),
            pl.BlockSpec((BN, HH), lambda i: (i, 0)),
            pl.BlockSpec((BN, CW), lambda i: (i, 0)),
            pl.BlockSpec((BN, H), lambda i: (i, 0)),
        ],
        out_specs=[
            pl.BlockSpec((BN, HH), lambda i: (i, 0)),
            pl.BlockSpec((BN, HH), lambda i: (i, 0)),
        ],
        out_shape=[
            jax.ShapeDtypeStruct((NP, HH), jnp.float32),
            jax.ShapeDtypeStruct((NP, HH), jnp.float32),
        ],
    )(s2L, s2R, cnt, h2)


def _seg_sum():
    out_type = [
        jax.ShapeDtypeStruct((NP, HH), jnp.float32),
        jax.ShapeDtypeStruct((NP, HH), jnp.float32),
    ]

    scratch = (
        [pltpu.VMEM((EBNIT * ECH,), jnp.int32)] * 2
        + [pltpu.VMEM((ECH,), jnp.int32)] * ERING
        + [pltpu.VMEM((ECH, HH), jnp.float32)] * ERING
        + [pltpu.VMEM((16, HH), jnp.float32)]
        + [pltpu.SemaphoreType.DMA] * ERING
        + [pltpu.SemaphoreType.DMA] * ERING
        + [pltpu.SemaphoreType.DMA]
        + [pltpu.VMEM_SHARED((NP, HH), jnp.float32)]
    )

    def body(src_hbm, dst_hbm, yl_hbm, yr_hbm, outL, outR, *scr):
        sidx_v, didx_v = scr[0], scr[1]
        dsml = list(scr[2:2 + ERING])
        rows = list(scr[2 + ERING:2 + 2 * ERING])
        zbuf = scr[2 + 2 * ERING]
        gsem = list(scr[3 + 2 * ERING:3 + 3 * ERING])
        ssem = list(scr[3 + 3 * ERING:3 + 4 * ERING])
        zsem = scr[3 + 4 * ERING]
        acc = scr[4 + 4 * ERING]

        c = lax.axis_index("c")
        s = lax.axis_index("s")
        row0 = s * RPT

        zeros16 = jnp.zeros((16,), jnp.float32)

        def zfill(r, carry):
            for j in range(HH // 16):
                zbuf[r, pl.ds(j * 16, 16)] = zeros16
            return carry
        lax.fori_loop(0, 16, zfill, 0)

        for k in range(RPT // 16):
            pltpu.async_copy(zbuf, acc.at[pl.ds(row0 + k * 16, 16)], zsem)
        for k in range(RPT // 16):
            pltpu.make_async_copy(zbuf, acc.at[pl.ds(row0 + k * 16, 16)],
                                  zsem).wait()

        plsc.subcore_barrier()

        ebase = s * EPT
        BWORDS = EBNIT * ECH

        def gather(b, q):
            for o in (0, 16, 24):
                dsml[b][pl.ds(o, 16)] = didx_v[pl.ds(q * ECH + o, 16)]

            @pl.when(c == 0)
            def _():
                pltpu.async_copy(yl_hbm.at[sidx_v.at[pl.ds(q * ECH, ECH)]],
                                 rows[b], gsem[b])

            @pl.when(c == 1)
            def _():
                pltpu.async_copy(yr_hbm.at[sidx_v.at[pl.ds(q * ECH, ECH)]],
                                 rows[b], gsem[b])

        def gather_wait(b):
            pltpu.make_async_copy(yl_hbm.at[sidx_v.at[pl.ds(0, ECH)]],
                                  rows[b], gsem[b]).wait()

        for blk in range(EBLK):
            bb = ebase + blk * BWORDS
            pltpu.async_copy(src_hbm.at[pl.ds(bb, BWORDS)], sidx_v, zsem)
            pltpu.async_copy(dst_hbm.at[pl.ds(bb, BWORDS)], didx_v, zsem)
            pltpu.make_async_copy(src_hbm.at[pl.ds(bb, BWORDS)], sidx_v,
                                  zsem).wait()
            pltpu.make_async_copy(dst_hbm.at[pl.ds(bb, BWORDS)], didx_v,
                                  zsem).wait()

            for b in range(ERING):
                gather(b, b)

            def eloop(g, carry):
                for b in range(ERING):
                    gather_wait(b)
                    pltpu.async_copy(rows[b], acc.at[dsml[b]], ssem[b],
                                     add=True)
                for b in range(ERING):
                    q = g * ERING + b
                    pltpu.make_async_copy(rows[b], acc.at[dsml[b]],
                                          ssem[b]).wait()

                    @pl.when(g < EBNIT // ERING - 1)
                    def _():
                        gather(b, q + ERING)
                return carry
            lax.fori_loop(0, EBNIT // ERING, eloop, 0)

        plsc.subcore_barrier()

        @pl.when(c == 0)
        def _():
            pltpu.sync_copy(acc.at[pl.ds(row0, RPT)], outL.at[pl.ds(row0, RPT)])

        @pl.when(c == 1)
        def _():
            pltpu.sync_copy(acc.at[pl.ds(row0, RPT)], outR.at[pl.ds(row0, RPT)])

    return pl.kernel(body, out_type=out_type, mesh=_MESH,
                     scratch_types=scratch)


_seg = _seg_sum()


def _sc_counts(dst):
    out_type = jax.ShapeDtypeStruct((NP, CW), jnp.float32)
    scratch = (
        [pltpu.VMEM((EBNIT * ECH,), jnp.int32)]
        + [pltpu.VMEM((ECH,), jnp.int32)] * ERING
        + [pltpu.VMEM((ECH, CW), jnp.float32)]
        + [pltpu.VMEM((16, CW), jnp.float32)]
        + [pltpu.SemaphoreType.DMA] * ERING
        + [pltpu.SemaphoreType.DMA]
        + [pltpu.VMEM_SHARED((NP, CW), jnp.float32)]
    )

    def body(dst_hbm, cnt_hbm, *scr):
        didx_v = scr[0]
        dsml = list(scr[1:1 + ERING])
        ones16 = scr[1 + ERING]
        zbuf16 = scr[2 + ERING]
        ssem = list(scr[3 + ERING:3 + 2 * ERING])
        zsem = scr[3 + 2 * ERING]
        cnt_sh = scr[4 + 2 * ERING]

        c = lax.axis_index("c")
        s = lax.axis_index("s")
        row0 = s * RPT
        zeros16 = jnp.zeros((16,), jnp.float32)
        ones16v = jnp.full((16,), 1.0, jnp.float32)

        @pl.when(c == 0)
        def _():
            def ofill(r, carry):
                for j in range(CW // 16):
                    ones16[r, pl.ds(j * 16, 16)] = ones16v
                return carry
            lax.fori_loop(0, ECH, ofill, 0)

            def zfill(r, carry):
                for j in range(CW // 16):
                    zbuf16[r, pl.ds(j * 16, 16)] = zeros16
                return carry
            lax.fori_loop(0, 16, zfill, 0)

            for k in range(RPT // 16):
                pltpu.async_copy(zbuf16, cnt_sh.at[pl.ds(row0 + k * 16, 16)],
                                 zsem)
            for k in range(RPT // 16):
                pltpu.make_async_copy(zbuf16,
                                      cnt_sh.at[pl.ds(row0 + k * 16, 16)],
                                      zsem).wait()

        plsc.subcore_barrier()

        @pl.when(c == 0)
        def _():
            ebase = s * EPT
            BWORDS = EBNIT * ECH

            def stage(b, q):
                for o in (0, 16, 24):
                    dsml[b][pl.ds(o, 16)] = didx_v[pl.ds(q * ECH + o, 16)]
                pltpu.async_copy(ones16, cnt_sh.at[dsml[b]], ssem[b],
                                 add=True)

            for blk in range(EBLK):
                bb = ebase + blk * BWORDS
                pltpu.async_copy(dst_hbm.at[pl.ds(bb, BWORDS)], didx_v, zsem)
                pltpu.make_async_copy(dst_hbm.at[pl.ds(bb, BWORDS)], didx_v,
                                      zsem).wait()

                for b in range(ERING):
                    stage(b, b)

                def eloop(g, carry):
                    for b in range(ERING):
                        q = g * ERING + b
                        pltpu.make_async_copy(ones16, cnt_sh.at[dsml[b]],
                                              ssem[b]).wait()

                        @pl.when(g < EBNIT // ERING - 1)
                        def _():
                            stage(b, q + ERING)
                    return carry
                lax.fori_loop(0, EBNIT // ERING, eloop, 0)

        plsc.subcore_barrier()

        @pl.when(c == 0)
        def _():
            pltpu.sync_copy(cnt_sh.at[pl.ds(row0, RPT)],
                            cnt_hbm.at[pl.ds(row0, RPT)])

    return pl.kernel(body, out_type=out_type, mesh=_MESH,
                     scratch_types=scratch)(dst)


def _sc_gather_pairs(lsrc2d, ldst2d, zL, zR):
    out_type = [jax.ShapeDtypeStruct((LP, HH), jnp.float32)] * 4
    scratch = (
        [pltpu.VMEM((PNIT, PCH), jnp.int32)] * 2
        + [pltpu.VMEM((PCH, HH), jnp.float32)] * (2 * PRING)
        + [pltpu.SemaphoreType.DMA] * PRING
        + [pltpu.SemaphoreType.DMA] * PRING
        + [pltpu.SemaphoreType.DMA]
    )

    def body(ls_hbm, ld_hbm, zl_hbm, zr_hbm,
             aL_hbm, aR_hbm, bL_hbm, bR_hbm, *scr):
        lsidx_v, ldidx_v = scr[0], scr[1]
        abuf = list(scr[2:2 + PRING])
        bbuf = list(scr[2 + PRING:2 + 2 * PRING])
        gsem = list(scr[2 + 2 * PRING:2 + 3 * PRING])
        wsem = list(scr[2 + 3 * PRING:2 + 4 * PRING])
        zsem = scr[2 + 4 * PRING]

        c = lax.axis_index("c")
        s = lax.axis_index("s")
        wbase = s * PPW

        pltpu.async_copy(ls_hbm.at[s], lsidx_v, zsem)
        pltpu.async_copy(ld_hbm.at[s], ldidx_v, zsem)
        pltpu.make_async_copy(ls_hbm.at[s], lsidx_v, zsem).wait()
        pltpu.make_async_copy(ld_hbm.at[s], ldidx_v, zsem).wait()

        def gather_set(b, q):
            @pl.when(c == 0)
            def _():
                pltpu.async_copy(zl_hbm.at[lsidx_v.at[q]], abuf[b], gsem[b])
                pltpu.async_copy(zl_hbm.at[ldidx_v.at[q]], bbuf[b], gsem[b])

            @pl.when(c == 1)
            def _():
                pltpu.async_copy(zr_hbm.at[lsidx_v.at[q]], abuf[b], gsem[b])
                pltpu.async_copy(zr_hbm.at[ldidx_v.at[q]], bbuf[b], gsem[b])

        def gather_wait(b):
            for buf in (abuf[b], bbuf[b]):
                pltpu.make_async_copy(zl_hbm.at[lsidx_v.at[0]], buf,
                                      gsem[b]).wait()

        def write_set(b, q):
            dst = pl.ds(wbase + q * PCH, PCH)

            @pl.when(c == 0)
            def _():
                pltpu.async_copy(abuf[b], aL_hbm.at[dst], wsem[b])
                pltpu.async_copy(bbuf[b], bL_hbm.at[dst], wsem[b])

            @pl.when(c == 1)
            def _():
                pltpu.async_copy(abuf[b], aR_hbm.at[dst], wsem[b])
                pltpu.async_copy(bbuf[b], bR_hbm.at[dst], wsem[b])

        def write_wait(b):
            dst = pl.ds(wbase, PCH)
            pltpu.make_async_copy(abuf[b], aL_hbm.at[dst], wsem[b]).wait()
            pltpu.make_async_copy(bbuf[b], bL_hbm.at[dst], wsem[b]).wait()

        for b in range(PRING):
            gather_set(b, b)

        def ploop(g, carry):
            for b in range(PRING):
                q = g * PRING + b
                gather_wait(b)
                write_set(b, q)
                write_wait(b)

                @pl.when(g < PNIT // PRING - 1)
                def _():
                    gather_set(b, q + PRING)
            return carry
        lax.fori_loop(0, PNIT // PRING, ploop, 0)

    return pl.kernel(body, out_type=out_type, mesh=_MESH,
                     scratch_types=scratch)(lsrc2d, ldst2d, zL, zR)


DBL = 2048


def _tc_d(aL, aR, bL, bR):
    def body(aL_ref, aR_ref, bL_ref, bR_ref, out_ref):
        x = aL_ref[...] * bL_ref[...] + aR_ref[...] * bR_ref[...]
        out_ref[...] = jnp.sum(x, axis=1).reshape(DBL // 128, 128)

    out2d = pl.pallas_call(
        body,
        grid=(LP // DBL,),
        in_specs=[pl.BlockSpec((DBL, HH), lambda i: (i, 0))] * 4,
        out_specs=pl.BlockSpec((DBL // 128, 128), lambda i: (i, 0)),
        out_shape=jax.ShapeDtypeStruct((LP // 128, 128), jnp.float32),
    )(aL, aR, bL, bR)
    return out2d.reshape(LP)[:L]


def kernel(edge_index, edge_label_index, emb, W1l, b1l, W1r, W2l, b2l, W2r):
    src = edge_index[0]
    dst = edge_index[1]
    lsrc = edge_label_index[0].reshape(NS, PNIT, PCH)
    ldst = edge_label_index[1].reshape(NS, PNIT, PCH)

    emb_p = jnp.concatenate(
        [emb, jnp.zeros((NP - N, H), emb.dtype)], axis=0)
    b1 = b1l.reshape(1, H)
    b2 = b2l.reshape(1, H)

    y1L, y1R, h1 = _tc_a(emb_p, W1l, W1r, b1)
    cnt = _sc_counts(dst)
    s1L, s1R = _seg(src, dst, y1L, y1R)
    y2L, y2R, h2 = _tc_b(s1L, s1R, cnt, h1, W2l, W2r, b2)
    s2L, s2R = _seg(src, dst, y2L, y2R)
    zL, zR = _tc_c(s2L, s2R, cnt, h2)
    aL, aR, bL, bR = _sc_gather_pairs(lsrc, ldst, zL, zR)
    return _tc_d(aL, aR, bL, bR)


# speedup vs baseline: 5.0249x; 1.0093x over previous
import functools

import jax
import jax.numpy as jnp
from jax import lax
from jax.experimental import pallas as pl
from jax.experimental.pallas import tpu as pltpu
from jax.experimental.pallas import tpu_sc as plsc

N = 10000
H = 256
HH = 128
HHP = 64
E = 160000
L = 160000

NC = 2
NS = 16
NW = NC * NS

NP = 10240
RPT = NP // NS

EPT = E // NS
ECH = 40
EBLK = 5
EBNIT = 50
ERING = 5
CW = 128

LP = 163840
PPW = L // NS
PCH = 40
PNIT = PPW // PCH
PRING = 5

BN = 1024

_MESH = plsc.VectorSubcoreMesh(
    core_axis_name="c", subcore_axis_name="s", num_cores=NC, num_subcores=NS)


def _dotT(x, w):
    return lax.dot_general(x, w, (((1,), (1,)), ((), ())),
                           preferred_element_type=jnp.float32)


def _tc_a(emb_p, W1l, W1r, b1l):
    def body(x_ref, wl_ref, wr_ref, b_ref, yL_ref, yR_ref, h_ref):
        x = x_ref[...]
        y = _dotT(x, wl_ref[...])
        yL_ref[...] = y[:, :HH]
        yR_ref[...] = y[:, HH:]
        h_ref[...] = _dotT(x, wr_ref[...]) + b_ref[...]

    return pl.pallas_call(
        body,
        grid=(NP // BN,),
        in_specs=[
            pl.BlockSpec((BN, H), lambda i: (i, 0)),
            pl.BlockSpec((H, H), lambda i: (0, 0)),
            pl.BlockSpec((H, H), lambda i: (0, 0)),
            pl.BlockSpec((1, H), lambda i: (0, 0)),
        ],
        out_specs=[
            pl.BlockSpec((BN, HH), lambda i: (i, 0)),
            pl.BlockSpec((BN, HH), lambda i: (i, 0)),
            pl.BlockSpec((BN, H), lambda i: (i, 0)),
        ],
        out_shape=[
            jax.ShapeDtypeStruct((NP, HH), jnp.float32),
            jax.ShapeDtypeStruct((NP, HH), jnp.float32),
            jax.ShapeDtypeStruct((NP, H), jnp.float32),
        ],
    )(emb_p, W1l, W1r, b1l)


def _tc_b(s1L, s1R, cnt0, cnt1, h1, W2l, W2r, b2l):
    def body(sL_ref, sR_ref, c0_ref, c1_ref, h_ref, wl_ref, wr_ref, b_ref,
             yL_ref, yR_ref, h2_ref):
        inv = 1.0 / jnp.maximum(c0_ref[...][:, :1] + c1_ref[...][:, :1], 1.0)
        s = jnp.concatenate([sL_ref[...], sR_ref[...]], axis=1)
        x = jnp.maximum(s * inv + h_ref[...], 0.0)
        y = _dotT(x, wl_ref[...])
        yL_ref[...] = y[:, :HH]
        yR_ref[...] = y[:, HH:]
        h2_ref[...] = _dotT(x, wr_ref[...]) + b_ref[...]

    return pl.pallas_call(
        body,
        grid=(NP // BN,),
        in_specs=[
            pl.BlockSpec((BN, HH), lambda i: (i, 0)),
            pl.BlockSpec((BN, HH), lambda i: (i, 0)),
            pl.BlockSpec((BN, CW), lambda i: (i, 0)),
            pl.BlockSpec((BN, CW), lambda i: (i, 0)),
            pl.BlockSpec((BN, H), lambda i: (i, 0)),
            pl.BlockSpec((H, H), lambda i: (0, 0)),
            pl.BlockSpec((H, H), lambda i: (0, 0)),
            pl.BlockSpec((1, H), lambda i: (0, 0)),
        ],
        out_specs=[
            pl.BlockSpec((BN, HH), lambda i: (i, 0)),
            pl.BlockSpec((BN, HH), lambda i: (i, 0)),
            pl.BlockSpec((BN, H), lambda i: (i, 0)),
        ],
        out_shape=[
            jax.ShapeDtypeStruct((NP, HH), jnp.float32),
            jax.ShapeDtypeStruct((NP, HH), jnp.float32),
            jax.ShapeDtypeStruct((NP, H), jnp.float32),
        ],
    )(s1L, s1R, cnt0, cnt1, h1, W2l, W2r, b2l)


def _tc_c(s2L, s2R, cnt0, cnt1, h2):
    def body(sL_ref, sR_ref, c0_ref, c1_ref, h_ref, zL_ref, zR_ref):
        inv = 1.0 / jnp.maximum(c0_ref[...][:, :1] + c1_ref[...][:, :1], 1.0)
        h = h_ref[...]
        zL_ref[...] = sL_ref[...] * inv + h[:, :HH]
        zR_ref[...] = sR_ref[...] * inv + h[:, HH:]

    return pl.pallas_call(
        body,
        grid=(NP // BN,),
        in_specs=[
            pl.BlockSpec((BN, HH), lambda i: (i, 0)),
            pl.BlockSpec((BN, HH), lambda i: (i, 0)),
            pl.BlockSpec((BN, CW), lambda i: (i, 0)),
            pl.BlockSpec((BN, CW), lambda i: (i, 0)),
            pl.BlockSpec((BN, H), lambda i: (i, 0)),
        ],
        out_specs=[
            pl.BlockSpec((BN, HH), lambda i: (i, 0)),
            pl.BlockSpec((BN, HH), lambda i: (i, 0)),
        ],
        out_shape=[
            jax.ShapeDtypeStruct((NP, HH), jnp.float32),
            jax.ShapeDtypeStruct((NP, HH), jnp.float32),
        ],
    )(s2L, s2R, cnt0, cnt1, h2)


def _seg_sum():
    out_type = [
        jax.ShapeDtypeStruct((NP, HH), jnp.float32),
        jax.ShapeDtypeStruct((NP, HH), jnp.float32),
    ]

    scratch = (
        [pltpu.VMEM((EBNIT * ECH,), jnp.int32)] * 2
        + [pltpu.VMEM((ECH,), jnp.int32)] * ERING
        + [pltpu.VMEM((ECH, HH), jnp.float32)] * ERING
        + [pltpu.VMEM((16, HH), jnp.float32)]
        + [pltpu.SemaphoreType.DMA] * ERING
        + [pltpu.SemaphoreType.DMA] * ERING
        + [pltpu.SemaphoreType.DMA]
        + [pltpu.VMEM_SHARED((NP, HH), jnp.float32)]
    )

    def body(src_hbm, dst_hbm, yl_hbm, yr_hbm, outL, outR, *scr):
        sidx_v, didx_v = scr[0], scr[1]
        dsml = list(scr[2:2 + ERING])
        rows = list(scr[2 + ERING:2 + 2 * ERING])
        zbuf = scr[2 + 2 * ERING]
        gsem = list(scr[3 + 2 * ERING:3 + 3 * ERING])
        ssem = list(scr[3 + 3 * ERING:3 + 4 * ERING])
        zsem = scr[3 + 4 * ERING]
        acc = scr[4 + 4 * ERING]

        c = lax.axis_index("c")
        s = lax.axis_index("s")
        row0 = s * RPT

        zeros16 = jnp.zeros((16,), jnp.float32)

        def zfill(r, carry):
            for j in range(HH // 16):
                zbuf[r, pl.ds(j * 16, 16)] = zeros16
            return carry
        lax.fori_loop(0, 16, zfill, 0)

        for k in range(RPT // 16):
            pltpu.async_copy(zbuf, acc.at[pl.ds(row0 + k * 16, 16)], zsem)
        for k in range(RPT // 16):
            pltpu.make_async_copy(zbuf, acc.at[pl.ds(row0 + k * 16, 16)],
                                  zsem).wait()

        plsc.subcore_barrier()

        ebase = s * EPT
        BWORDS = EBNIT * ECH

        def gather(b, q):
            for o in (0, 16, 24):
                dsml[b][pl.ds(o, 16)] = didx_v[pl.ds(q * ECH + o, 16)]

            @pl.when(c == 0)
            def _():
                pltpu.async_copy(yl_hbm.at[sidx_v.at[pl.ds(q * ECH, ECH)]],
                                 rows[b], gsem[b])

            @pl.when(c == 1)
            def _():
                pltpu.async_copy(yr_hbm.at[sidx_v.at[pl.ds(q * ECH, ECH)]],
                                 rows[b], gsem[b])

        def gather_wait(b):
            pltpu.make_async_copy(yl_hbm.at[sidx_v.at[pl.ds(0, ECH)]],
                                  rows[b], gsem[b]).wait()

        for blk in range(EBLK):
            bb = ebase + blk * BWORDS
            pltpu.async_copy(src_hbm.at[pl.ds(bb, BWORDS)], sidx_v, zsem)
            pltpu.async_copy(dst_hbm.at[pl.ds(bb, BWORDS)], didx_v, zsem)
            pltpu.make_async_copy(src_hbm.at[pl.ds(bb, BWORDS)], sidx_v,
                                  zsem).wait()
            pltpu.make_async_copy(dst_hbm.at[pl.ds(bb, BWORDS)], didx_v,
                                  zsem).wait()

            for b in range(ERING):
                gather(b, b)

            def eloop(g, carry):
                for b in range(ERING):
                    gather_wait(b)
                    pltpu.async_copy(rows[b], acc.at[dsml[b]], ssem[b],
                                     add=True)
                for b in range(ERING):
                    q = g * ERING + b
                    pltpu.make_async_copy(rows[b], acc.at[dsml[b]],
                                          ssem[b]).wait()

                    @pl.when(g < EBNIT // ERING - 1)
                    def _():
                        gather(b, q + ERING)
                return carry
            lax.fori_loop(0, EBNIT // ERING, eloop, 0)

        plsc.subcore_barrier()

        @pl.when(c == 0)
        def _():
            pltpu.sync_copy(acc.at[pl.ds(row0, RPT)], outL.at[pl.ds(row0, RPT)])

        @pl.when(c == 1)
        def _():
            pltpu.sync_copy(acc.at[pl.ds(row0, RPT)], outR.at[pl.ds(row0, RPT)])

    return pl.kernel(body, out_type=out_type, mesh=_MESH,
                     scratch_types=scratch)


_seg = _seg_sum()


def _sc_counts(dst):
    out_type = [jax.ShapeDtypeStruct((NP, CW), jnp.float32)] * 2
    CPT = E // NW
    CBLK = 5
    CBNIT = CPT // (CBLK * ECH)
    scratch = (
        [pltpu.VMEM((CBNIT * ECH,), jnp.int32)]
        + [pltpu.VMEM((ECH,), jnp.int32)] * ERING
        + [pltpu.VMEM((ECH, CW), jnp.float32)]
        + [pltpu.VMEM((16, CW), jnp.float32)]
        + [pltpu.SemaphoreType.DMA] * ERING
        + [pltpu.SemaphoreType.DMA]
        + [pltpu.VMEM_SHARED((NP, CW), jnp.float32)]
    )

    def body(dst_hbm, cnt0_hbm, cnt1_hbm, *scr):
        didx_v = scr[0]
        dsml = list(scr[1:1 + ERING])
        ones_b = scr[1 + ERING]
        zbuf16 = scr[2 + ERING]
        ssem = list(scr[3 + ERING:3 + 2 * ERING])
        zsem = scr[3 + 2 * ERING]
        cnt_sh = scr[4 + 2 * ERING]

        c = lax.axis_index("c")
        s = lax.axis_index("s")
        row0 = s * RPT
        zeros16 = jnp.zeros((16,), jnp.float32)
        ones16v = jnp.full((16,), 1.0, jnp.float32)

        def ofill(r, carry):
            for j in range(CW // 16):
                ones_b[r, pl.ds(j * 16, 16)] = ones16v
            return carry
        lax.fori_loop(0, ECH, ofill, 0)

        def zfill(r, carry):
            for j in range(CW // 16):
                zbuf16[r, pl.ds(j * 16, 16)] = zeros16
            return carry
        lax.fori_loop(0, 16, zfill, 0)

        for k in range(RPT // 16):
            pltpu.async_copy(zbuf16, cnt_sh.at[pl.ds(row0 + k * 16, 16)],
                             zsem)
        for k in range(RPT // 16):
            pltpu.make_async_copy(zbuf16,
                                  cnt_sh.at[pl.ds(row0 + k * 16, 16)],
                                  zsem).wait()

        plsc.subcore_barrier()

        ebase = (c * NS + s) * CPT
        BWORDS = CBNIT * ECH

        def stage(b, q):
            for o in (0, 16, 24):
                dsml[b][pl.ds(o, 16)] = didx_v[pl.ds(q * ECH + o, 16)]
            pltpu.async_copy(ones_b, cnt_sh.at[dsml[b]], ssem[b],
                             add=True)

        for blk in range(CBLK):
            bb = ebase + blk * BWORDS
            pltpu.async_copy(dst_hbm.at[pl.ds(bb, BWORDS)], didx_v, zsem)
            pltpu.make_async_copy(dst_hbm.at[pl.ds(bb, BWORDS)], didx_v,
                                  zsem).wait()

            for b in range(ERING):
                stage(b, b)

            def eloop(g, carry):
                for b in range(ERING):
                    q = g * ERING + b
                    pltpu.make_async_copy(ones_b, cnt_sh.at[dsml[b]],
                                          ssem[b]).wait()

                    @pl.when(g < CBNIT // ERING - 1)
                    def _():
                        stage(b, q + ERING)
                return carry
            lax.fori_loop(0, CBNIT // ERING, eloop, 0)

        plsc.subcore_barrier()

        @pl.when(c == 0)
        def _():
            pltpu.sync_copy(cnt_sh.at[pl.ds(row0, RPT)],
                            cnt0_hbm.at[pl.ds(row0, RPT)])

        @pl.when(c == 1)
        def _():
            pltpu.sync_copy(cnt_sh.at[pl.ds(row0, RPT)],
                            cnt1_hbm.at[pl.ds(row0, RPT)])

    return pl.kernel(body, out_type=out_type, mesh=_MESH,
                     scratch_types=scratch)(dst)


def _sc_gather_pairs(lsrc, ldst, zL, zR):
    out_type = [jax.ShapeDtypeStruct((LP, HH), jnp.float32)] * 4
    scratch = (
        [pltpu.VMEM((PPW,), jnp.int32)] * 2
        + [pltpu.VMEM((PCH, HH), jnp.float32)] * (2 * PRING)
        + [pltpu.SemaphoreType.DMA] * PRING
        + [pltpu.SemaphoreType.DMA] * PRING
        + [pltpu.SemaphoreType.DMA]
    )

    def body(ls_hbm, ld_hbm, zl_hbm, zr_hbm,
             aL_hbm, aR_hbm, bL_hbm, bR_hbm, *scr):
        lsidx_v, ldidx_v = scr[0], scr[1]
        abuf = list(scr[2:2 + PRING])
        bbuf = list(scr[2 + PRING:2 + 2 * PRING])
        gsem = list(scr[2 + 2 * PRING:2 + 3 * PRING])
        wsem = list(scr[2 + 3 * PRING:2 + 4 * PRING])
        zsem = scr[2 + 4 * PRING]

        c = lax.axis_index("c")
        s = lax.axis_index("s")
        wbase = s * PPW

        pltpu.async_copy(ls_hbm.at[pl.ds(wbase, PPW)], lsidx_v, zsem)
        pltpu.async_copy(ld_hbm.at[pl.ds(wbase, PPW)], ldidx_v, zsem)
        pltpu.make_async_copy(ls_hbm.at[pl.ds(wbase, PPW)], lsidx_v,
                              zsem).wait()
        pltpu.make_async_copy(ld_hbm.at[pl.ds(wbase, PPW)], ldidx_v,
                              zsem).wait()

        def gather_set(b, q):
            qs = pl.ds(q * PCH, PCH)

            @pl.when(c == 0)
            def _():
                pltpu.async_copy(zl_hbm.at[lsidx_v.at[qs]], abuf[b], gsem[b])
                pltpu.async_copy(zl_hbm.at[ldidx_v.at[qs]], bbuf[b], gsem[b])

            @pl.when(c == 1)
            def _():
                pltpu.async_copy(zr_hbm.at[lsidx_v.at[qs]], abuf[b], gsem[b])
                pltpu.async_copy(zr_hbm.at[ldidx_v.at[qs]], bbuf[b], gsem[b])

        def gather_wait(b):
            for buf in (abuf[b], bbuf[b]):
                pltpu.make_async_copy(zl_hbm.at[lsidx_v.at[pl.ds(0, PCH)]],
                                      buf, gsem[b]).wait()

        def write_set(b, q):
            dst = pl.ds(wbase + q * PCH, PCH)

            @pl.when(c == 0)
            def _():
                pltpu.async_copy(abuf[b], aL_hbm.at[dst], wsem[b])
                pltpu.async_copy(bbuf[b], bL_hbm.at[dst], wsem[b])

            @pl.when(c == 1)
            def _():
                pltpu.async_copy(abuf[b], aR_hbm.at[dst], wsem[b])
                pltpu.async_copy(bbuf[b], bR_hbm.at[dst], wsem[b])

        def write_wait(b):
            dst = pl.ds(wbase, PCH)
            pltpu.make_async_copy(abuf[b], aL_hbm.at[dst], wsem[b]).wait()
            pltpu.make_async_copy(bbuf[b], bL_hbm.at[dst], wsem[b]).wait()

        for b in range(PRING):
            gather_set(b, b)

        def ploop(g, carry):
            for b in range(PRING):
                q = g * PRING + b
                gather_wait(b)
                write_set(b, q)
                write_wait(b)

                @pl.when(g < PNIT // PRING - 1)
                def _():
                    gather_set(b, q + PRING)
            return carry
        lax.fori_loop(0, PNIT // PRING, ploop, 0)

    return pl.kernel(body, out_type=out_type, mesh=_MESH,
                     scratch_types=scratch)(lsrc, ldst, zL, zR)


DBL = 2048


def _tc_d(aL, aR, bL, bR):
    def body(aL_ref, aR_ref, bL_ref, bR_ref, out_ref):
        x = aL_ref[...] * bL_ref[...] + aR_ref[...] * bR_ref[...]
        out_ref[...] = jnp.sum(x, axis=1).reshape(DBL // 128, 128)

    out2d = pl.pallas_call(
        body,
        grid=(LP // DBL,),
        in_specs=[pl.BlockSpec((DBL, HH), lambda i: (i, 0))] * 4,
        out_specs=pl.BlockSpec((DBL // 128, 128), lambda i: (i, 0)),
        out_shape=jax.ShapeDtypeStruct((LP // 128, 128), jnp.float32),
    )(aL, aR, bL, bR)
    return out2d.reshape(LP)[:L]


def kernel(edge_index, edge_label_index, emb, W1l, b1l, W1r, W2l, b2l, W2r):
    src = edge_index[0]
    dst = edge_index[1]
    lsrc = edge_label_index[0]
    ldst = edge_label_index[1]

    emb_p = jnp.concatenate(
        [emb, jnp.zeros((NP - N, H), emb.dtype)], axis=0)
    b1 = b1l.reshape(1, H)
    b2 = b2l.reshape(1, H)

    y1L, y1R, h1 = _tc_a(emb_p, W1l, W1r, b1)
    cnt0, cnt1 = _sc_counts(dst)
    s1L, s1R = _seg(src, dst, y1L, y1R)
    y2L, y2R, h2 = _tc_b(s1L, s1R, cnt0, cnt1, h1, W2l, W2r, b2)
    s2L, s2R = _seg(src, dst, y2L, y2R)
    zL, zR = _tc_c(s2L, s2R, cnt0, cnt1, h2)
    aL, aR, bL, bR = _sc_gather_pairs(lsrc, ldst, zL, zR)
    return _tc_d(aL, aR, bL, bR)


# speedup vs baseline: 5.0258x; 1.0002x over previous
import jax
import jax.numpy as jnp
from jax import lax
from jax.experimental import pallas as pl
from jax.experimental.pallas import tpu as pltpu
from jax.experimental.pallas import tpu_sc as plsc

N = 10000
H = 256
HH = 128
HHP = 64
E = 160000
L = 160000

NC = 2
NS = 16
NW = NC * NS

NP = 10240
RPT = NP // NS

EPT = E // NS
ECH = 40
EBLK = 5
EBNIT = 50
ERING = 5
CW = 128

LP = 163840
PPW = L // NS
PCH = 40
PNIT = PPW // PCH
PRING = 5

BN = 1024

_MESH = plsc.VectorSubcoreMesh(
    core_axis_name="c", subcore_axis_name="s", num_cores=NC, num_subcores=NS)


def _dotT(x, w):
    return lax.dot_general(x, w, (((1,), (1,)), ((), ())),
                           preferred_element_type=jnp.float32)


def _tc_a(emb_p, W1l, W1r, b1l):
    def body(x_ref, wl_ref, wr_ref, b_ref, yL_ref, yR_ref, h_ref):
        x = x_ref[...]
        y = _dotT(x, wl_ref[...])
        yL_ref[...] = y[:, :HH]
        yR_ref[...] = y[:, HH:]
        h_ref[...] = _dotT(x, wr_ref[...]) + b_ref[...]

    return pl.pallas_call(
        body,
        grid=(NP // BN,),
        in_specs=[
            pl.BlockSpec((BN, H), lambda i: (i, 0)),
            pl.BlockSpec((H, H), lambda i: (0, 0)),
            pl.BlockSpec((H, H), lambda i: (0, 0)),
            pl.BlockSpec((1, H), lambda i: (0, 0)),
        ],
        out_specs=[
            pl.BlockSpec((BN, HH), lambda i: (i, 0)),
            pl.BlockSpec((BN, HH), lambda i: (i, 0)),
            pl.BlockSpec((BN, H), lambda i: (i, 0)),
        ],
        out_shape=[
            jax.ShapeDtypeStruct((NP, HH), jnp.float32),
            jax.ShapeDtypeStruct((NP, HH), jnp.float32),
            jax.ShapeDtypeStruct((NP, H), jnp.float32),
        ],
    )(emb_p, W1l, W1r, b1l)


def _tc_b(s1L, s1R, cnt0, cnt1, h1, W2l, W2r, b2l):
    def body(sL_ref, sR_ref, c0_ref, c1_ref, h_ref, wl_ref, wr_ref, b_ref,
             yL_ref, yR_ref, h2_ref):
        inv = 1.0 / jnp.maximum(c0_ref[...][:, :1] + c1_ref[...][:, :1], 1.0)
        s = jnp.concatenate([sL_ref[...], sR_ref[...]], axis=1)
        x = jnp.maximum(s * inv + h_ref[...], 0.0)
        y = _dotT(x, wl_ref[...])
        yL_ref[...] = y[:, :HH]
        yR_ref[...] = y[:, HH:]
        h2_ref[...] = _dotT(x, wr_ref[...]) + b_ref[...]

    return pl.pallas_call(
        body,
        grid=(NP // BN,),
        in_specs=[
            pl.BlockSpec((BN, HH), lambda i: (i, 0)),
            pl.BlockSpec((BN, HH), lambda i: (i, 0)),
            pl.BlockSpec((BN, CW), lambda i: (i, 0)),
            pl.BlockSpec((BN, CW), lambda i: (i, 0)),
            pl.BlockSpec((BN, H), lambda i: (i, 0)),
            pl.BlockSpec((H, H), lambda i: (0, 0)),
            pl.BlockSpec((H, H), lambda i: (0, 0)),
            pl.BlockSpec((1, H), lambda i: (0, 0)),
        ],
        out_specs=[
            pl.BlockSpec((BN, HH), lambda i: (i, 0)),
            pl.BlockSpec((BN, HH), lambda i: (i, 0)),
            pl.BlockSpec((BN, H), lambda i: (i, 0)),
        ],
        out_shape=[
            jax.ShapeDtypeStruct((NP, HH), jnp.float32),
            jax.ShapeDtypeStruct((NP, HH), jnp.float32),
            jax.ShapeDtypeStruct((NP, H), jnp.float32),
        ],
    )(s1L, s1R, cnt0, cnt1, h1, W2l, W2r, b2l)


def _tc_c(s2L, s2R, cnt0, cnt1, h2):
    def body(sL_ref, sR_ref, c0_ref, c1_ref, h_ref, zL_ref, zR_ref):
        inv = 1.0 / jnp.maximum(c0_ref[...][:, :1] + c1_ref[...][:, :1], 1.0)
        h = h_ref[...]
        zL_ref[...] = sL_ref[...] * inv + h[:, :HH]
        zR_ref[...] = sR_ref[...] * inv + h[:, HH:]

    return pl.pallas_call(
        body,
        grid=(NP // BN,),
        in_specs=[
            pl.BlockSpec((BN, HH), lambda i: (i, 0)),
            pl.BlockSpec((BN, HH), lambda i: (i, 0)),
            pl.BlockSpec((BN, CW), lambda i: (i, 0)),
            pl.BlockSpec((BN, CW), lambda i: (i, 0)),
            pl.BlockSpec((BN, H), lambda i: (i, 0)),
        ],
        out_specs=[
            pl.BlockSpec((BN, HH), lambda i: (i, 0)),
            pl.BlockSpec((BN, HH), lambda i: (i, 0)),
        ],
        out_shape=[
            jax.ShapeDtypeStruct((NP, HH), jnp.float32),
            jax.ShapeDtypeStruct((NP, HH), jnp.float32),
        ],
    )(s2L, s2R, cnt0, cnt1, h2)


def _seg_sum():
    out_type = [
        jax.ShapeDtypeStruct((NP, HH), jnp.float32),
        jax.ShapeDtypeStruct((NP, HH), jnp.float32),
    ]

    scratch = (
        [pltpu.VMEM((EBNIT * ECH,), jnp.int32)] * 2
        + [pltpu.VMEM((ECH,), jnp.int32)] * ERING
        + [pltpu.VMEM((ECH, HH), jnp.float32)] * ERING
        + [pltpu.VMEM((16, HH), jnp.float32)]
        + [pltpu.SemaphoreType.DMA] * ERING
        + [pltpu.SemaphoreType.DMA] * ERING
        + [pltpu.SemaphoreType.DMA]
        + [pltpu.VMEM_SHARED((NP, HH), jnp.float32)]
    )

    def body(src_hbm, dst_hbm, yl_hbm, yr_hbm, outL, outR, *scr):
        sidx_v, didx_v = scr[0], scr[1]
        dsml = list(scr[2:2 + ERING])
        rows = list(scr[2 + ERING:2 + 2 * ERING])
        zbuf = scr[2 + 2 * ERING]
        gsem = list(scr[3 + 2 * ERING:3 + 3 * ERING])
        ssem = list(scr[3 + 3 * ERING:3 + 4 * ERING])
        zsem = scr[3 + 4 * ERING]
        acc = scr[4 + 4 * ERING]

        c = lax.axis_index("c")
        s = lax.axis_index("s")
        row0 = s * RPT

        zeros16 = jnp.zeros((16,), jnp.float32)

        def zfill(r, carry):
            for j in range(HH // 16):
                zbuf[r, pl.ds(j * 16, 16)] = zeros16
            return carry
        lax.fori_loop(0, 16, zfill, 0)

        for k in range(RPT // 16):
            pltpu.async_copy(zbuf, acc.at[pl.ds(row0 + k * 16, 16)], zsem)
        for k in range(RPT // 16):
            pltpu.make_async_copy(zbuf, acc.at[pl.ds(row0 + k * 16, 16)],
                                  zsem).wait()

        plsc.subcore_barrier()

        ebase = s * EPT
        BWORDS = EBNIT * ECH

        def gather(b, q):
            for o in (0, 16, 24):
                dsml[b][pl.ds(o, 16)] = didx_v[pl.ds(q * ECH + o, 16)]

            @pl.when(c == 0)
            def _():
                pltpu.async_copy(yl_hbm.at[sidx_v.at[pl.ds(q * ECH, ECH)]],
                                 rows[b], gsem[b])

            @pl.when(c == 1)
            def _():
                pltpu.async_copy(yr_hbm.at[sidx_v.at[pl.ds(q * ECH, ECH)]],
                                 rows[b], gsem[b])

        def gather_wait(b):
            pltpu.make_async_copy(yl_hbm.at[sidx_v.at[pl.ds(0, ECH)]],
                                  rows[b], gsem[b]).wait()

        for blk in range(EBLK):
            bb = ebase + blk * BWORDS
            pltpu.async_copy(src_hbm.at[pl.ds(bb, BWORDS)], sidx_v, zsem)
            pltpu.async_copy(dst_hbm.at[pl.ds(bb, BWORDS)], didx_v, zsem)
            pltpu.make_async_copy(src_hbm.at[pl.ds(bb, BWORDS)], sidx_v,
                                  zsem).wait()
            pltpu.make_async_copy(dst_hbm.at[pl.ds(bb, BWORDS)], didx_v,
                                  zsem).wait()

            for b in range(ERING):
                gather(b, b)

            def eloop(g, carry):
                for b in range(ERING):
                    gather_wait(b)
                    pltpu.async_copy(rows[b], acc.at[dsml[b]], ssem[b],
                                     add=True)
                for b in range(ERING):
                    q = g * ERING + b
                    pltpu.make_async_copy(rows[b], acc.at[dsml[b]],
                                          ssem[b]).wait()

                    @pl.when(g < EBNIT // ERING - 1)
                    def _():
                        gather(b, q + ERING)
                return carry
            lax.fori_loop(0, EBNIT // ERING, eloop, 0)

        plsc.subcore_barrier()

        @pl.when(c == 0)
        def _():
            pltpu.sync_copy(acc.at[pl.ds(row0, RPT)], outL.at[pl.ds(row0, RPT)])

        @pl.when(c == 1)
        def _():
            pltpu.sync_copy(acc.at[pl.ds(row0, RPT)], outR.at[pl.ds(row0, RPT)])

    return pl.kernel(body, out_type=out_type, mesh=_MESH,
                     scratch_types=scratch)


_seg = _seg_sum()


def _sc_counts(dst):
    out_type = [jax.ShapeDtypeStruct((NP, CW), jnp.float32)] * 2
    CPT = E // NW
    CBLK = 5
    CBNIT = CPT // (CBLK * ECH)
    scratch = (
        [pltpu.VMEM((CBNIT * ECH,), jnp.int32)]
        + [pltpu.VMEM((ECH,), jnp.int32)] * ERING
        + [pltpu.VMEM((ECH, CW), jnp.float32)]
        + [pltpu.VMEM((16, CW), jnp.float32)]
        + [pltpu.SemaphoreType.DMA] * ERING
        + [pltpu.SemaphoreType.DMA]
        + [pltpu.VMEM_SHARED((NP, CW), jnp.float32)]
    )

    def body(dst_hbm, cnt0_hbm, cnt1_hbm, *scr):
        didx_v = scr[0]
        dsml = list(scr[1:1 + ERING])
        ones_b = scr[1 + ERING]
        zbuf16 = scr[2 + ERING]
        ssem = list(scr[3 + ERING:3 + 2 * ERING])
        zsem = scr[3 + 2 * ERING]
        cnt_sh = scr[4 + 2 * ERING]

        c = lax.axis_index("c")
        s = lax.axis_index("s")
        row0 = s * RPT
        zeros16 = jnp.zeros((16,), jnp.float32)
        ones16v = jnp.full((16,), 1.0, jnp.float32)

        def ofill(r, carry):
            for j in range(CW // 16):
                ones_b[r, pl.ds(j * 16, 16)] = ones16v
            return carry
        lax.fori_loop(0, ECH, ofill, 0)

        def zfill(r, carry):
            for j in range(CW // 16):
                zbuf16[r, pl.ds(j * 16, 16)] = zeros16
            return carry
        lax.fori_loop(0, 16, zfill, 0)

        for k in range(RPT // 16):
            pltpu.async_copy(zbuf16, cnt_sh.at[pl.ds(row0 + k * 16, 16)],
                             zsem)
        for k in range(RPT // 16):
            pltpu.make_async_copy(zbuf16,
                                  cnt_sh.at[pl.ds(row0 + k * 16, 16)],
                                  zsem).wait()

        plsc.subcore_barrier()

        ebase = (c * NS + s) * CPT
        BWORDS = CBNIT * ECH

        def stage(b, q):
            for o in (0, 16, 24):
                dsml[b][pl.ds(o, 16)] = didx_v[pl.ds(q * ECH + o, 16)]
            pltpu.async_copy(ones_b, cnt_sh.at[dsml[b]], ssem[b],
                             add=True)

        for blk in range(CBLK):
            bb = ebase + blk * BWORDS
            pltpu.async_copy(dst_hbm.at[pl.ds(bb, BWORDS)], didx_v, zsem)
            pltpu.make_async_copy(dst_hbm.at[pl.ds(bb, BWORDS)], didx_v,
                                  zsem).wait()

            for b in range(ERING):
                stage(b, b)

            def eloop(g, carry):
                for b in range(ERING):
                    q = g * ERING + b
                    pltpu.make_async_copy(ones_b, cnt_sh.at[dsml[b]],
                                          ssem[b]).wait()

                    @pl.when(g < CBNIT // ERING - 1)
                    def _():
                        stage(b, q + ERING)
                return carry
            lax.fori_loop(0, CBNIT // ERING, eloop, 0)

        plsc.subcore_barrier()

        @pl.when(c == 0)
        def _():
            pltpu.sync_copy(cnt_sh.at[pl.ds(row0, RPT)],
                            cnt0_hbm.at[pl.ds(row0, RPT)])

        @pl.when(c == 1)
        def _():
            pltpu.sync_copy(cnt_sh.at[pl.ds(row0, RPT)],
                            cnt1_hbm.at[pl.ds(row0, RPT)])

    return pl.kernel(body, out_type=out_type, mesh=_MESH,
                     scratch_types=scratch)(dst)


def _sc_gather_pairs(lsrc, ldst, zL, zR):
    out_type = [jax.ShapeDtypeStruct((LP, HH), jnp.float32)] * 4
    scratch = (
        [pltpu.VMEM((PPW,), jnp.int32)] * 2
        + [pltpu.VMEM((PCH, HH), jnp.float32)] * (2 * PRING)
        + [pltpu.SemaphoreType.DMA] * PRING
        + [pltpu.SemaphoreType.DMA] * PRING
        + [pltpu.SemaphoreType.DMA]
    )

    def body(ls_hbm, ld_hbm, zl_hbm, zr_hbm,
             aL_hbm, aR_hbm, bL_hbm, bR_hbm, *scr):
        lsidx_v, ldidx_v = scr[0], scr[1]
        abuf = list(scr[2:2 + PRING])
        bbuf = list(scr[2 + PRING:2 + 2 * PRING])
        gsem = list(scr[2 + 2 * PRING:2 + 3 * PRING])
        wsem = list(scr[2 + 3 * PRING:2 + 4 * PRING])
        zsem = scr[2 + 4 * PRING]

        c = lax.axis_index("c")
        s = lax.axis_index("s")
        wbase = s * PPW

        pltpu.async_copy(ls_hbm.at[pl.ds(wbase, PPW)], lsidx_v, zsem)
        pltpu.async_copy(ld_hbm.at[pl.ds(wbase, PPW)], ldidx_v, zsem)
        pltpu.make_async_copy(ls_hbm.at[pl.ds(wbase, PPW)], lsidx_v,
                              zsem).wait()
        pltpu.make_async_copy(ld_hbm.at[pl.ds(wbase, PPW)], ldidx_v,
                              zsem).wait()

        def gather_set(b, q):
            qs = pl.ds(q * PCH, PCH)

            @pl.when(c == 0)
            def _():
                pltpu.async_copy(zl_hbm.at[lsidx_v.at[qs]], abuf[b], gsem[b])
                pltpu.async_copy(zl_hbm.at[ldidx_v.at[qs]], bbuf[b], gsem[b])

            @pl.when(c == 1)
            def _():
                pltpu.async_copy(zr_hbm.at[lsidx_v.at[qs]], abuf[b], gsem[b])
                pltpu.async_copy(zr_hbm.at[ldidx_v.at[qs]], bbuf[b], gsem[b])

        def gather_wait(b):
            for buf in (abuf[b], bbuf[b]):
                pltpu.make_async_copy(zl_hbm.at[lsidx_v.at[pl.ds(0, PCH)]],
                                      buf, gsem[b]).wait()

        def write_set(b, q):
            dst = pl.ds(wbase + q * PCH, PCH)

            @pl.when(c == 0)
            def _():
                pltpu.async_copy(abuf[b], aL_hbm.at[dst], wsem[b])
                pltpu.async_copy(bbuf[b], bL_hbm.at[dst], wsem[b])

            @pl.when(c == 1)
            def _():
                pltpu.async_copy(abuf[b], aR_hbm.at[dst], wsem[b])
                pltpu.async_copy(bbuf[b], bR_hbm.at[dst], wsem[b])

        def write_wait(b):
            dst = pl.ds(wbase, PCH)
            pltpu.make_async_copy(abuf[b], aL_hbm.at[dst], wsem[b]).wait()
            pltpu.make_async_copy(bbuf[b], bL_hbm.at[dst], wsem[b]).wait()

        for b in range(PRING):
            gather_set(b, b)

        def ploop(g, carry):
            for b in range(PRING):
                q = g * PRING + b
                gather_wait(b)
                write_set(b, q)
                write_wait(b)

                @pl.when(g < PNIT // PRING - 1)
                def _():
                    gather_set(b, q + PRING)
            return carry
        lax.fori_loop(0, PNIT // PRING, ploop, 0)

    return pl.kernel(body, out_type=out_type, mesh=_MESH,
                     scratch_types=scratch)(lsrc, ldst, zL, zR)


DBL = 2048


def _tc_d(aL, aR, bL, bR):
    def body(aL_ref, aR_ref, bL_ref, bR_ref, out_ref):
        x = aL_ref[...] * bL_ref[...] + aR_ref[...] * bR_ref[...]
        out_ref[...] = jnp.sum(x, axis=1).reshape(DBL // 128, 128)

    out2d = pl.pallas_call(
        body,
        grid=(LP // DBL,),
        in_specs=[pl.BlockSpec((DBL, HH), lambda i: (i, 0))] * 4,
        out_specs=pl.BlockSpec((DBL // 128, 128), lambda i: (i, 0)),
        out_shape=jax.ShapeDtypeStruct((LP // 128, 128), jnp.float32),
    )(aL, aR, bL, bR)
    return out2d.reshape(LP)[:L]


def kernel(edge_index, edge_label_index, emb, W1l, b1l, W1r, W2l, b2l, W2r):
    src = edge_index[0]
    dst = edge_index[1]
    lsrc = edge_label_index[0]
    ldst = edge_label_index[1]

    emb_p = jnp.concatenate(
        [emb, jnp.zeros((NP - N, H), emb.dtype)], axis=0)
    b1 = b1l.reshape(1, H)
    b2 = b2l.reshape(1, H)

    y1L, y1R, h1 = _tc_a(emb_p, W1l, W1r, b1)
    cnt0, cnt1 = _sc_counts(dst)
    s1L, s1R = _seg(src, dst, y1L, y1R)
    y2L, y2R, h2 = _tc_b(s1L, s1R, cnt0, cnt1, h1, W2l, W2r, b2)
    s2L, s2R = _seg(src, dst, y2L, y2R)
    zL, zR = _tc_c(s2L, s2R, cnt0, cnt1, h2)
    aL, aR, bL, bR = _sc_gather_pairs(lsrc, ldst, zL, zR)
    return _tc_d(aL, aR, bL, bR)
